# baseline (speedup 1.0000x reference)
import functools

import jax
import jax.numpy as jnp
from jax import lax
from jax.experimental import pallas as pl
from jax.experimental.pallas import tpu as pltpu
from jax.experimental.pallas import tpu_sc as plsc

N = 10000
E = 320000
IN_CH = 128
HID = 128
ZDIM = 32
NUM_CLASSES = 2

NUM_TILES = 16
NUM_WORKERS = 32
E_PAD = 327680
ROWS_PER_TILE = 632
N_PAD = N + 8

ROW_BLK = 2000

_CFG = {
    16: dict(chunk=128, cpt=80, k=4, groups=1),
    32: dict(chunk=128, cpt=80, k=4, groups=1),
    128: dict(chunk=64, cpt=160, k=2, groups=2),
}


def _sc_mesh():
    return plsc.VectorSubcoreMesh(core_axis_name="c", subcore_axis_name="s")


@functools.cache
def _make_agg(w):
    cfg = _CFG[w]
    chunk, cpt, k, groups = cfg["chunk"], cfg["cpt"], cfg["k"], cfg["groups"]
    cpg = cpt // groups
    nb = cpg // k

    @functools.partial(
        pl.kernel,
        mesh=_sc_mesh(),
        compiler_params=pltpu.CompilerParams(use_tc_tiling_on_sc=False),
        out_type=[jax.ShapeDtypeStruct((N, w), jnp.float32),
                  jax.ShapeDtypeStruct((N, w), jnp.float32)],
        scratch_types=[
            pltpu.VMEM_SHARED((N_PAD, w), jnp.float32),
            pltpu.VMEM((cpg, chunk), jnp.int32),
            pltpu.VMEM((cpg, chunk), jnp.int32),
            pltpu.VMEM((2 * k, chunk, w), jnp.float32),
            pltpu.SemaphoreType.DMA,
        ],
    )
    def agg(y, src2d, dst2d, s0, s1, acc, srcv, dstv, rb, g0):
        c = lax.axis_index("c")
        s = lax.axis_index("s")
        row0 = jnp.minimum(s * ROWS_PER_TILE, N - ROWS_PER_TILE)

        def fire_g(b, q, sem):
            return [pltpu.async_copy(y.at[srcv.at[b * k + i]],
                                     rb.at[q * k + i], sem)
                    for i in range(k)]

        def scat(b, q):
            for i in range(k):
                pltpu.sync_copy(rb.at[q * k + i],
                                acc.at[dstv.at[b * k + i]], add=True)

        def run(out_hbm, cbase):
            wbase = (cbase + s) * cpt
            pltpu.sync_copy(y.at[pl.ds(row0, ROWS_PER_TILE)],
                            acc.at[pl.ds(row0, ROWS_PER_TILE)])
            plsc.subcore_barrier()

            for g in range(groups):
                pltpu.sync_copy(src2d.at[pl.ds(wbase + g * cpg, cpg)], srcv)
                pltpu.sync_copy(dst2d.at[pl.ds(wbase + g * cpg, cpg)], dstv)

                def body(i, carry):
                    da = fire_g(i, 0, g0)
                    for d in da:
                        d.wait()
                    scat(i, 0)
                    return carry

                lax.fori_loop(0, nb, body, 0)

            plsc.subcore_barrier()
            pltpu.sync_copy(acc.at[pl.ds(row0, ROWS_PER_TILE)],
                            out_hbm.at[pl.ds(row0, ROWS_PER_TILE)])

        @pl.when(c == 0)
        def _():
            run(s0, 0)

        @pl.when(c == 1)
        def _():
            run(s1, NUM_TILES)

    return agg


def _act(a, kind):
    if kind == "relu":
        return jnp.maximum(a, 0.0)
    if kind == "sigmoid":
        return jax.nn.sigmoid(a)
    return a


def _row_spec(width):
    return pl.BlockSpec((ROW_BLK, width), lambda i: (i, 0))


def _full_spec(shape):
    return pl.BlockSpec(shape, lambda i: (0, 0))


def _dinv(d0_ref, d1_ref):
    return lax.rsqrt(d0_ref[...][:, :1] + d1_ref[...][:, :1] - 1.0)


def _split_out(v, refs):
    fc = refs[0].shape[1]
    for i, ref in enumerate(refs):
        ref[...] = v[:, i * fc:(i + 1) * fc]


def _nparts(f):
    return 2 if f > HID else 1


def _pre_body(x_ref, d0_ref, d1_ref, w_ref, *y_refs):
    y = jnp.dot(x_ref[...], w_ref[...], preferred_element_type=jnp.float32)
    _split_out(y * _dinv(d0_ref, d1_ref), y_refs)


def _pre(x, degs, w):
    f = w.shape[1]
    np_ = _nparts(f)
    fc = f // np_
    return pl.pallas_call(
        _pre_body,
        grid=(N // ROW_BLK,),
        in_specs=[_row_spec(x.shape[1]), _row_spec(16), _row_spec(16),
                  _full_spec(w.shape)],
        out_specs=[_row_spec(fc)] * np_,
        out_shape=[jax.ShapeDtypeStruct((N, fc), jnp.float32)] * np_,
    )(x, degs[0], degs[1], w)


def _comb_body(s0_ref, s1_ref, yp_ref, d0_ref, d1_ref, b_ref, w_ref,
               *y_refs, kind):
    dinv = _dinv(d0_ref, d1_ref)
    sfull = s0_ref[...] + s1_ref[...] - yp_ref[...]
    inp = _act(sfull * dinv + b_ref[...], kind)
    y = jnp.dot(inp, w_ref[...], preferred_element_type=jnp.float32)
    _split_out(y * dinv, y_refs)


def _comb(s_pair, y_prev, degs, b, w, kind):
    fprev = y_prev.shape[1]
    f = w.shape[1]
    np_ = _nparts(f)
    fc = f // np_
    return pl.pallas_call(
        functools.partial(_comb_body, kind=kind),
        grid=(N // ROW_BLK,),
        in_specs=[_row_spec(fprev)] * 3 + [_row_spec(16), _row_spec(16),
                                           _full_spec((1, fprev)),
                                           _full_spec(w.shape)],
        out_specs=[_row_spec(fc)] * np_,
        out_shape=[jax.ShapeDtypeStruct((N, fc), jnp.float32)] * np_,
    )(s_pair[0], s_pair[1], y_prev, degs[0], degs[1], b.reshape(1, fprev), w)


def _act_out_body(*refs, kind, nin):
    triples, rest = refs[:3 * nin], refs[3 * nin:]
    d0_ref, d1_ref, b_ref, o_ref = rest
    dinv = _dinv(d0_ref, d1_ref)
    parts = [triples[3 * i][...] + triples[3 * i + 1][...]
             - triples[3 * i + 2][...] for i in range(nin)]
    sfull = parts[0] if nin == 1 else jnp.concatenate(parts, axis=1)
    o_ref[...] = _act(sfull * dinv + b_ref[...], kind)


def _act_out(triples, degs, b, kind):
    f = sum(t[0].shape[1] for t in triples)
    flat = [a for t in triples for a in t]
    return pl.pallas_call(
        functools.partial(_act_out_body, kind=kind, nin=len(triples)),
        grid=(N // ROW_BLK,),
        in_specs=[_row_spec(a.shape[1]) for a in flat]
        + [_row_spec(16), _row_spec(16), _full_spec((1, f))],
        out_specs=_row_spec(f),
        out_shape=jax.ShapeDtypeStruct((N, f), jnp.float32),
    )(*flat, degs[0], degs[1], b.reshape(1, f))


def _zpre_body(mu_ref, si_ref, e_ref, d0_ref, d1_ref, w_ref, *y_refs):
    z = mu_ref[...] + si_ref[...] * e_ref[...]
    y = jnp.dot(z, w_ref[...], preferred_element_type=jnp.float32)
    _split_out(y * _dinv(d0_ref, d1_ref), y_refs)


def _zpre(mu, si, e, degs, w):
    f = w.shape[1]
    np_ = _nparts(f)
    fc = f // np_
    return pl.pallas_call(
        _zpre_body,
        grid=(N // ROW_BLK,),
        in_specs=[_row_spec(ZDIM), _row_spec(ZDIM), _row_spec(ZDIM),
                  _row_spec(16), _row_spec(16), _full_spec(w.shape)],
        out_specs=[_row_spec(fc)] * np_,
        out_shape=[jax.ShapeDtypeStruct((N, fc), jnp.float32)] * np_,
    )(mu, si, e, degs[0], degs[1], w)


def kernel(x, edge_index, W_in, b_in, W_mu0, b_mu0, W_mu, b_mu, W_si0, b_si0,
           W_si, b_si, W_zh, b_zh, W_rec0, b_rec0, W_out, b_out):
    src = edge_index[0].astype(jnp.int32)
    dst = edge_index[1].astype(jnp.int32)
    pad = E_PAD - E
    src1 = jnp.concatenate([src, jnp.zeros((pad,), jnp.int32)])
    dst1 = jnp.concatenate([dst, jnp.full((pad,), N, jnp.int32)])
    idx128 = (src1.reshape(-1, 128), dst1.reshape(-1, 128))
    idx64 = (src1.reshape(-1, 64), dst1.reshape(-1, 64))

    agg128 = _make_agg(128)
    agg32 = _make_agg(32)
    agg16 = _make_agg(16)

    ones16 = jnp.ones((N, 16), jnp.float32)
    degs = agg16(ones16, *idx128)

    y1 = _pre(x, degs, W_in)[0]
    s_in = agg128(y1, *idx64)
    y2 = _comb(s_in, y1, degs, b_in, W_mu0, "relu")[0]
    s_mu0 = agg128(y2, *idx64)
    y3 = _comb(s_mu0, y2, degs, b_mu0, W_mu, "relu")[0]
    s_mu = agg32(y3, *idx128)
    mu = _act_out([(s_mu[0], s_mu[1], y3)], degs, b_mu, "relu")
    y4 = _comb(s_in, y1, degs, b_in, W_si0, "relu")[0]
    s_si0 = agg128(y4, *idx64)
    y5 = _comb(s_si0, y4, degs, b_si0, W_si, "relu")[0]
    s_si = agg32(y5, *idx128)
    si = _act_out([(s_si[0], s_si[1], y5)], degs, b_si, "sigmoid")
    e = jax.random.normal(jax.random.key(1), (N, ZDIM), jnp.float32)
    y6 = _zpre(mu, si, e, degs, W_zh)[0]
    s_zh = agg128(y6, *idx64)
    y7 = _comb(s_zh, y6, degs, b_zh, W_rec0, "none")[0]
    s_rec = agg128(y7, *idx64)
    y8a, y8b = _comb(s_rec, y7, degs, b_rec0, W_out, "relu")
    s_a = agg128(y8a, *idx64)
    s_b = agg128(y8b, *idx64)
    logits = _act_out([(s_a[0], s_a[1], y8a), (s_b[0], s_b[1], y8b)],
                      degs, b_out, "none")
    logits = logits.reshape(-1, NUM_CLASSES)
    return (logits, mu, si)


# speedup vs baseline: 1.4758x; 1.4758x over previous
import functools

import jax
import jax.numpy as jnp
from jax import lax
from jax.experimental import pallas as pl
from jax.experimental.pallas import tpu as pltpu
from jax.experimental.pallas import tpu_sc as plsc

N = 10000
E = 320000
IN_CH = 128
HID = 128
ZDIM = 32
NUM_CLASSES = 2

NUM_TILES = 16
CHUNK = 128
NCHUNKS = 2560
E_PAD = NCHUNKS * CHUNK
CPT_ALL = 160
CPT_HALF = 80
ROWS_PER_TILE = 632
N_PAD = N + 8
RING = 5

ROW_BLK = 2000


def _sc_mesh():
    return plsc.VectorSubcoreMesh(core_axis_name="c", subcore_axis_name="s")


def _agg_scratch(fc, cpt):
    return [
        pltpu.VMEM_SHARED((N_PAD, fc), jnp.float32),
        pltpu.VMEM((cpt, CHUNK), jnp.int32),
        pltpu.VMEM((cpt, CHUNK), jnp.int32),
        pltpu.VMEM((RING, CHUNK, fc), jnp.float32),
    ] + [pltpu.SemaphoreType.DMA] * RING


def _agg_run(y, out, acc, srcv, dstv, rb, sems, src2d, dst2d, wbase, s, cpt):
    row0 = jnp.minimum(s * ROWS_PER_TILE, N - ROWS_PER_TILE)
    base = wbase + s * cpt
    pltpu.sync_copy(src2d.at[pl.ds(base, cpt)], srcv)
    pltpu.sync_copy(dst2d.at[pl.ds(base, cpt)], dstv)
    pltpu.sync_copy(y.at[pl.ds(row0, ROWS_PER_TILE)],
                    acc.at[pl.ds(row0, ROWS_PER_TILE)])
    plsc.subcore_barrier()

    def body(i, carry):
        b = i * RING
        gs = [pltpu.async_copy(y.at[srcv.at[b + j]], rb.at[j], sems[j])
              for j in range(RING)]
        ss = []
        for j in range(RING):
            gs[j].wait()
            ss.append(pltpu.async_copy(rb.at[j], acc.at[dstv.at[b + j]],
                                       sems[j], add=True))
        for d in ss:
            d.wait()
        return carry

    lax.fori_loop(0, cpt // RING, body, 0)
    plsc.subcore_barrier()
    pltpu.sync_copy(acc.at[pl.ds(row0, ROWS_PER_TILE)],
                    out.at[pl.ds(row0, ROWS_PER_TILE)])


@functools.partial(
    pl.kernel,
    mesh=_sc_mesh(),
    compiler_params=pltpu.CompilerParams(use_tc_tiling_on_sc=False),
    out_type=[jax.ShapeDtypeStruct((N, HID // 2), jnp.float32),
              jax.ShapeDtypeStruct((N, HID // 2), jnp.float32)],
    scratch_types=_agg_scratch(HID // 2, CPT_ALL),
)
def _agg_col(y_l, y_r, src2d, dst2d, s_l, s_r, acc, srcv, dstv, rb, *sems):
    c = lax.axis_index("c")
    s = lax.axis_index("s")

    @pl.when(c == 0)
    def _():
        _agg_run(y_l, s_l, acc, srcv, dstv, rb, sems, src2d, dst2d,
                 0, s, CPT_ALL)

    @pl.when(c == 1)
    def _():
        _agg_run(y_r, s_r, acc, srcv, dstv, rb, sems, src2d, dst2d,
                 0, s, CPT_ALL)


@functools.cache
def _make_agg_edge(w):
    @functools.partial(
        pl.kernel,
        mesh=_sc_mesh(),
        compiler_params=pltpu.CompilerParams(use_tc_tiling_on_sc=False),
        out_type=[jax.ShapeDtypeStruct((N, w), jnp.float32),
                  jax.ShapeDtypeStruct((N, w), jnp.float32)],
        scratch_types=_agg_scratch(w, CPT_HALF),
    )
    def agg(y, src2d, dst2d, s0, s1, acc, srcv, dstv, rb, *sems):
        c = lax.axis_index("c")
        s = lax.axis_index("s")

        @pl.when(c == 0)
        def _():
            _agg_run(y, s0, acc, srcv, dstv, rb, sems, src2d, dst2d,
                     0, s, CPT_HALF)

        @pl.when(c == 1)
        def _():
            _agg_run(y, s1, acc, srcv, dstv, rb, sems, src2d, dst2d,
                     NUM_TILES * CPT_HALF, s, CPT_HALF)

    return agg


def _act(a, kind):
    if kind == "relu":
        return jnp.maximum(a, 0.0)
    if kind == "sigmoid":
        return jax.nn.sigmoid(a)
    return a


def _row_spec(width):
    return pl.BlockSpec((ROW_BLK, width), lambda i: (i, 0))


def _full_spec(shape):
    return pl.BlockSpec(shape, lambda i: (0, 0))


def _dinv(d0_ref, d1_ref):
    return lax.rsqrt(d0_ref[...][:, :1] + d1_ref[...][:, :1] - 1.0)


def _split_out(v, refs):
    fc = refs[0].shape[1]
    for i, ref in enumerate(refs):
        ref[...] = v[:, i * fc:(i + 1) * fc]


def _nparts(f):
    return f // 64 if f >= HID else 1


def _pre_body(x_ref, d0_ref, d1_ref, w_ref, *y_refs):
    y = jnp.dot(x_ref[...], w_ref[...], preferred_element_type=jnp.float32)
    _split_out(y * _dinv(d0_ref, d1_ref), y_refs)


def _pre(x, degs, w):
    f = w.shape[1]
    np_ = _nparts(f)
    return pl.pallas_call(
        _pre_body,
        grid=(N // ROW_BLK,),
        in_specs=[_row_spec(x.shape[1]), _row_spec(16), _row_spec(16),
                  _full_spec(w.shape)],
        out_specs=[_row_spec(f // np_)] * np_,
        out_shape=[jax.ShapeDtypeStruct((N, f // np_), jnp.float32)] * np_,
    )(x, degs[0], degs[1], w)


def _comb_body(sl_ref, sr_ref, d0_ref, d1_ref, b_ref, w_ref, *y_refs, kind):
    dinv = _dinv(d0_ref, d1_ref)
    sfull = jnp.concatenate([sl_ref[...], sr_ref[...]], axis=1)
    inp = _act(sfull * dinv + b_ref[...], kind)
    y = jnp.dot(inp, w_ref[...], preferred_element_type=jnp.float32)
    _split_out(y * dinv, y_refs)


def _comb(s_pair, degs, b, w, kind):
    fprev = 2 * s_pair[0].shape[1]
    f = w.shape[1]
    np_ = _nparts(f)
    return pl.pallas_call(
        functools.partial(_comb_body, kind=kind),
        grid=(N // ROW_BLK,),
        in_specs=[_row_spec(s_pair[0].shape[1])] * 2
        + [_row_spec(16), _row_spec(16), _full_spec((1, fprev)),
           _full_spec(w.shape)],
        out_specs=[_row_spec(f // np_)] * np_,
        out_shape=[jax.ShapeDtypeStruct((N, f // np_), jnp.float32)] * np_,
    )(s_pair[0], s_pair[1], degs[0], degs[1], b.reshape(1, fprev), w)


def _edge_out_body(s0_ref, s1_ref, yp_ref, d0_ref, d1_ref, b_ref, o_ref, *,
                   kind):
    dinv = _dinv(d0_ref, d1_ref)
    sfull = s0_ref[...] + s1_ref[...] - yp_ref[...]
    o_ref[...] = _act(sfull * dinv + b_ref[...], kind)


def _edge_out(s_pair, y_prev, degs, b, kind):
    f = y_prev.shape[1]
    return pl.pallas_call(
        functools.partial(_edge_out_body, kind=kind),
        grid=(N // ROW_BLK,),
        in_specs=[_row_spec(f)] * 3 + [_row_spec(16), _row_spec(16),
                                       _full_spec((1, f))],
        out_specs=_row_spec(f),
        out_shape=jax.ShapeDtypeStruct((N, f), jnp.float32),
    )(s_pair[0], s_pair[1], y_prev, degs[0], degs[1], b.reshape(1, f))


def _quad_out_body(s0, s1, s2, s3, d0_ref, d1_ref, b_ref, o_ref, *, kind):
    dinv = _dinv(d0_ref, d1_ref)
    sfull = jnp.concatenate([s0[...], s1[...], s2[...], s3[...]], axis=1)
    o_ref[...] = _act(sfull * dinv + b_ref[...], kind)


def _quad_out(parts, degs, b, kind):
    f = sum(p.shape[1] for p in parts)
    return pl.pallas_call(
        functools.partial(_quad_out_body, kind=kind),
        grid=(N // ROW_BLK,),
        in_specs=[_row_spec(p.shape[1]) for p in parts]
        + [_row_spec(16), _row_spec(16), _full_spec((1, f))],
        out_specs=_row_spec(f),
        out_shape=jax.ShapeDtypeStruct((N, f), jnp.float32),
    )(*parts, degs[0], degs[1], b.reshape(1, f))


def _zpre_body(mu_ref, si_ref, e_ref, d0_ref, d1_ref, w_ref, *y_refs):
    z = mu_ref[...] + si_ref[...] * e_ref[...]
    y = jnp.dot(z, w_ref[...], preferred_element_type=jnp.float32)
    _split_out(y * _dinv(d0_ref, d1_ref), y_refs)


def _zpre(mu, si, e, degs, w):
    f = w.shape[1]
    np_ = _nparts(f)
    return pl.pallas_call(
        _zpre_body,
        grid=(N // ROW_BLK,),
        in_specs=[_row_spec(ZDIM), _row_spec(ZDIM), _row_spec(ZDIM),
                  _row_spec(16), _row_spec(16), _full_spec(w.shape)],
        out_specs=[_row_spec(f // np_)] * np_,
        out_shape=[jax.ShapeDtypeStruct((N, f // np_), jnp.float32)] * np_,
    )(mu, si, e, degs[0], degs[1], w)


def kernel(x, edge_index, W_in, b_in, W_mu0, b_mu0, W_mu, b_mu, W_si0, b_si0,
           W_si, b_si, W_zh, b_zh, W_rec0, b_rec0, W_out, b_out):
    src = edge_index[0].astype(jnp.int32)
    dst = edge_index[1].astype(jnp.int32)
    pad = E_PAD - E
    src2d = jnp.concatenate(
        [src, jnp.zeros((pad,), jnp.int32)]).reshape(NCHUNKS, CHUNK)
    dst2d = jnp.concatenate(
        [dst, jnp.full((pad,), N, jnp.int32)]).reshape(NCHUNKS, CHUNK)

    agg32 = _make_agg_edge(32)
    agg16 = _make_agg_edge(16)

    ones16 = jnp.ones((N, 16), jnp.float32)
    degs = agg16(ones16, src2d, dst2d)

    y1 = _pre(x, degs, W_in)
    s_in = _agg_col(y1[0], y1[1], src2d, dst2d)
    y2 = _comb(s_in, degs, b_in, W_mu0, "relu")
    s_mu0 = _agg_col(y2[0], y2[1], src2d, dst2d)
    y3 = _comb(s_mu0, degs, b_mu0, W_mu, "relu")[0]
    s_mu = agg32(y3, src2d, dst2d)
    mu = _edge_out(s_mu, y3, degs, b_mu, "relu")
    y4 = _comb(s_in, degs, b_in, W_si0, "relu")
    s_si0 = _agg_col(y4[0], y4[1], src2d, dst2d)
    y5 = _comb(s_si0, degs, b_si0, W_si, "relu")[0]
    s_si = agg32(y5, src2d, dst2d)
    si = _edge_out(s_si, y5, degs, b_si, "sigmoid")
    e = jax.random.normal(jax.random.key(1), (N, ZDIM), jnp.float32)
    y6 = _zpre(mu, si, e, degs, W_zh)
    s_zh = _agg_col(y6[0], y6[1], src2d, dst2d)
    y7 = _comb(s_zh, degs, b_zh, W_rec0, "none")
    s_rec = _agg_col(y7[0], y7[1], src2d, dst2d)
    y8 = _comb(s_rec, degs, b_rec0, W_out, "relu")
    s_a = _agg_col(y8[0], y8[1], src2d, dst2d)
    s_b = _agg_col(y8[2], y8[3], src2d, dst2d)
    logits = _quad_out([s_a[0], s_a[1], s_b[0], s_b[1]], degs, b_out, "none")
    logits = logits.reshape(-1, NUM_CLASSES)
    return (logits, mu, si)
